# baseline (speedup 1.0000x reference)
import jax
import jax.numpy as jnp
from jax.experimental import pallas as pl
from jax.experimental.pallas import tpu as pltpu


def _gcn_body(x_ref, w_ref, b_ref, adja_ref, adjb_ref, out_ref, support_ref):
    i = pl.program_id(0)
    bm = adja_ref.shape[0]

    @pl.when(i == 0)
    def _():
        support_ref[...] = jnp.dot(
            x_ref[...], w_ref[...], preferred_element_type=jnp.float32
        ).astype(jnp.bfloat16)

    sup = support_ref[...]
    out_ref[0:bm, :] = (
        jnp.dot(adja_ref[...].astype(jnp.bfloat16), sup,
                preferred_element_type=jnp.float32)
        + b_ref[...]
    )
    out_ref[bm:2 * bm, :] = (
        jnp.dot(adjb_ref[...].astype(jnp.bfloat16), sup,
                preferred_element_type=jnp.float32)
        + b_ref[...]
    )


def kernel(inputs, adj, weight, bias):
    n, d_in = inputs.shape
    d_out = weight.shape[1]
    bm = 200
    bias2 = bias.reshape(1, d_out)
    grid = (n // (2 * bm),)
    return pl.pallas_call(
        _gcn_body,
        grid=grid,
        in_specs=[
            pl.BlockSpec((n, d_in), lambda i: (0, 0)),
            pl.BlockSpec((d_in, d_out), lambda i: (0, 0)),
            pl.BlockSpec((1, d_out), lambda i: (0, 0)),
            pl.BlockSpec((bm, n), lambda i: (2 * i, 0)),
            pl.BlockSpec((bm, n), lambda i: (2 * i + 1, 0)),
        ],
        out_specs=pl.BlockSpec((2 * bm, d_out), lambda i: (i, 0)),
        out_shape=jax.ShapeDtypeStruct((n, d_out), jnp.float32),
        scratch_shapes=[pltpu.VMEM((n, d_out), jnp.bfloat16)],
    )(inputs, weight, bias2, adj, adj)
